# Initial kernel scaffold; baseline (speedup 1.0000x reference)
import jax
import jax.numpy as jnp
from jax.experimental import pallas as pl
from jax.experimental.pallas import tpu as pltpu

N = 10000
E = 160000
B = 8
NEG = 0.2


def _gat(x, src, dst, W, al, ar, b, H, D):
    h = (x @ W).reshape(N, H, D)
    el = jnp.sum(h * al[None, :, :], axis=-1)
    er = jnp.sum(h * ar[None, :, :], axis=-1)
    e = jax.nn.leaky_relu(el[src] + er[dst], NEG)
    m = jax.ops.segment_max(e, dst, num_segments=N)
    m = jnp.where(jnp.isfinite(m), m, 0.0)
    ee = jnp.exp(e - m[dst])
    denom = jax.ops.segment_sum(ee, dst, num_segments=N)
    alpha = ee / jnp.maximum(denom[dst], 1e-9)
    out = jax.ops.segment_sum(alpha[:, :, None] * h[src], dst, num_segments=N)
    return out + b.reshape(1, H, D)


def _readout_mlp_body(hlig_ref, hrec_ref, ohl_ref, ohr_ref,
                      w1_ref, b1_ref, w2_ref, b2_ref, out_ref):
    def seg_max(h, oh):
        rows = []
        for g in range(B):
            m = oh[:, g][:, None] > 0
            v = jnp.max(jnp.where(m, h, -jnp.inf), axis=0)
            rows.append(jnp.where(v > -1e30, v, 0.0))
        return jnp.stack(rows)

    sl = seg_max(hlig_ref[...], ohl_ref[...])
    sr = seg_max(hrec_ref[...], ohr_ref[...])
    hcat = jnp.concatenate([sl, sr], axis=1)
    z = jnp.maximum(jnp.dot(hcat, w1_ref[...],
                            preferred_element_type=jnp.float32) + b1_ref[...], 0.0)
    z = jnp.maximum(jnp.dot(z, w2_ref[...],
                            preferred_element_type=jnp.float32) + b2_ref[...], 0.0)
    out_ref[...] = z


def _readout_mlp(hlig, hrec, ohl, ohr, lin1_w, lin1_b, lin2_w, lin2_b):
    return pl.pallas_call(
        _readout_mlp_body,
        out_shape=jax.ShapeDtypeStruct((B, 1), jnp.float32),
    )(hlig, hrec, ohl, ohr, lin1_w, lin1_b.reshape(1, -1), lin2_w,
      lin2_b.reshape(1, 1))


def kernel(lig_x, rec_x, lig_edge_index, rec_edge_index, lig_graph_ids,
           rec_graph_ids, W1l, al1l, ar1l, b1l, W2l, al2l, ar2l, b2l,
           W1r, al1r, ar1r, b1r, W2r, al2r, ar2r, b2r,
           lin1_w, lin1_b, lin2_w, lin2_b):
    ls = lig_edge_index[0].astype(jnp.int32)
    ld = lig_edge_index[1].astype(jnp.int32)
    rs = rec_edge_index[0].astype(jnp.int32)
    rd = rec_edge_index[1].astype(jnp.int32)

    hlig = jax.nn.relu(_gat(lig_x, ls, ld, W1l, al1l, ar1l, b1l, 10, 64)).sum(1)
    hlig = jax.nn.relu(_gat(hlig, ls, ld, W2l, al2l, ar2l, b2l, 1, 128))
    hlig = hlig.reshape(N, 128)
    hrec = jax.nn.relu(_gat(rec_x, rs, rd, W1r, al1r, ar1r, b1r, 10, 64)).sum(1)
    hrec = jax.nn.relu(_gat(hrec, rs, rd, W2r, al2r, ar2r, b2r, 1, 128))
    hrec = hrec.reshape(N, 128)

    ohl = jax.nn.one_hot(lig_graph_ids, B, dtype=jnp.float32)
    ohr = jax.nn.one_hot(rec_graph_ids, B, dtype=jnp.float32)
    out = _readout_mlp(hlig, hrec, ohl, ohr, lin1_w, lin1_b, lin2_w, lin2_b)
    return out.flatten()


# speedup vs baseline: 1.0020x; 1.0020x over previous
import jax
import jax.numpy as jnp
from jax.experimental import pallas as pl
from jax.experimental.pallas import tpu as pltpu

N = 10000
E = 160000
B = 8
NEG = 0.2


def _gat(x, src, dst, W, al, ar, b, H, D):
    h = (x @ W).reshape(N, H, D)
    el = jnp.sum(h * al[None, :, :], axis=-1)
    er = jnp.sum(h * ar[None, :, :], axis=-1)
    e = jax.nn.leaky_relu(el[src] + er[dst], NEG)
    m = jax.ops.segment_max(e, dst, num_segments=N)
    m = jnp.where(jnp.isfinite(m), m, 0.0)
    ee = jnp.exp(e - m[dst])
    denom = jax.ops.segment_sum(ee, dst, num_segments=N)
    alpha = ee / jnp.maximum(denom[dst], 1e-9)
    out = jax.ops.segment_sum(alpha[:, :, None] * h[src], dst, num_segments=N)
    return out + b.reshape(1, H, D)


_RB = 1000
_NB = N // _RB


def _readout_mlp_body(hlig_ref, hrec_ref, ohl_ref, ohr_ref,
                      w1_ref, b1_ref, w2_ref, b2_ref, out_ref,
                      accl_ref, accr_ref):
    i = pl.program_id(0)

    @pl.when(i == 0)
    def _init():
        accl_ref[...] = jnp.full((B, 128), -jnp.inf, jnp.float32)
        accr_ref[...] = jnp.full((B, 128), -jnp.inf, jnp.float32)

    def upd(acc_ref, h, oh):
        for g in range(B):
            m = oh[:, g][:, None] > 0
            v = jnp.max(jnp.where(m, h, -jnp.inf), axis=0)
            acc_ref[g, :] = jnp.maximum(acc_ref[g, :], v)

    upd(accl_ref, hlig_ref[...], ohl_ref[...])
    upd(accr_ref, hrec_ref[...], ohr_ref[...])

    @pl.when(i == _NB - 1)
    def _fin():
        sl = accl_ref[...]
        sl = jnp.where(sl > -1e30, sl, 0.0)
        sr = accr_ref[...]
        sr = jnp.where(sr > -1e30, sr, 0.0)
        hcat = jnp.concatenate([sl, sr], axis=1)
        z = jnp.maximum(jnp.dot(hcat, w1_ref[...],
                                preferred_element_type=jnp.float32)
                        + b1_ref[...], 0.0)
        z = jnp.maximum(jnp.dot(z, w2_ref[...],
                                preferred_element_type=jnp.float32)
                        + b2_ref[...], 0.0)
        out_ref[...] = z


def _readout_mlp(hlig, hrec, ohl, ohr, lin1_w, lin1_b, lin2_w, lin2_b):
    return pl.pallas_call(
        _readout_mlp_body,
        grid=(_NB,),
        in_specs=[
            pl.BlockSpec((_RB, 128), lambda i: (i, 0)),
            pl.BlockSpec((_RB, 128), lambda i: (i, 0)),
            pl.BlockSpec((_RB, B), lambda i: (i, 0)),
            pl.BlockSpec((_RB, B), lambda i: (i, 0)),
            pl.BlockSpec((256, 128), lambda i: (0, 0)),
            pl.BlockSpec((1, 128), lambda i: (0, 0)),
            pl.BlockSpec((128, 1), lambda i: (0, 0)),
            pl.BlockSpec((1, 1), lambda i: (0, 0)),
        ],
        out_specs=pl.BlockSpec((B, 1), lambda i: (0, 0)),
        out_shape=jax.ShapeDtypeStruct((B, 1), jnp.float32),
        scratch_shapes=[pltpu.VMEM((B, 128), jnp.float32),
                        pltpu.VMEM((B, 128), jnp.float32)],
    )(hlig, hrec, ohl, ohr, lin1_w, lin1_b.reshape(1, -1), lin2_w,
      lin2_b.reshape(1, 1))


def kernel(lig_x, rec_x, lig_edge_index, rec_edge_index, lig_graph_ids,
           rec_graph_ids, W1l, al1l, ar1l, b1l, W2l, al2l, ar2l, b2l,
           W1r, al1r, ar1r, b1r, W2r, al2r, ar2r, b2r,
           lin1_w, lin1_b, lin2_w, lin2_b):
    ls = lig_edge_index[0].astype(jnp.int32)
    ld = lig_edge_index[1].astype(jnp.int32)
    rs = rec_edge_index[0].astype(jnp.int32)
    rd = rec_edge_index[1].astype(jnp.int32)

    hlig = jax.nn.relu(_gat(lig_x, ls, ld, W1l, al1l, ar1l, b1l, 10, 64)).sum(1)
    hlig = jax.nn.relu(_gat(hlig, ls, ld, W2l, al2l, ar2l, b2l, 1, 128))
    hlig = hlig.reshape(N, 128)
    hrec = jax.nn.relu(_gat(rec_x, rs, rd, W1r, al1r, ar1r, b1r, 10, 64)).sum(1)
    hrec = jax.nn.relu(_gat(hrec, rs, rd, W2r, al2r, ar2r, b2r, 1, 128))
    hrec = hrec.reshape(N, 128)

    ohl = jax.nn.one_hot(lig_graph_ids, B, dtype=jnp.float32)
    ohr = jax.nn.one_hot(rec_graph_ids, B, dtype=jnp.float32)
    out = _readout_mlp(hlig, hrec, ohl, ohr, lin1_w, lin1_b, lin2_w, lin2_b)
    return out.flatten()


# speedup vs baseline: 4.1994x; 4.1911x over previous
import functools

import jax
import jax.numpy as jnp
from jax import lax
from jax.experimental import pallas as pl
from jax.experimental.pallas import tpu as pltpu
from jax.experimental.pallas import tpu_sc as plsc

N = 10000
E = 160000
B = 8
NEG = 0.2

NPT = 160
NR = 64
NPAD = NR * NPT
SEGE = 1024


def _sc_gat_edge(H, D, HDP):
    HD = H * D
    mesh = plsc.VectorSubcoreMesh(core_axis_name="c", subcore_axis_name="s")

    def body(htab, er16p, srcs, dsts, starts, bias, out,
             acc1, rows2, segs, segd, den1, erloc, idxs, startsv, biasv,
             sem_rows):
        wid = lax.axis_index("s") * 2 + lax.axis_index("c")
        pltpu.sync_copy(starts, startsv)
        pltpu.sync_copy(bias, biasv)
        iota = lax.iota(jnp.int32, 16)
        zero16 = iota * 0

        def rr_body(rr, _):
            r = wid * 2 + rr
            nbase = r * NPT

            s0 = jnp.max(plsc.load_gather(startsv, [zero16 + r]))
            s1 = jnp.max(plsc.load_gather(startsv, [zero16 + (r + 1)]))

            @plsc.parallel_loop(0, NPT * HD // 16, 1, unroll=8)
            def _zacc(i):
                acc1[pl.ds(i * 16, 16)] = jnp.zeros((16,), jnp.float32)

            @plsc.parallel_loop(0, NPT, 1, unroll=8)
            def _zden(i):
                den1[pl.ds(i * 16, 16)] = jnp.zeros((16,), jnp.float32)

            pltpu.sync_copy(er16p.at[pl.ds(nbase * 16, NPT * 16)], erloc)

            def proc(srcv, dstv, valid):
                idxs[...] = srcv
                cr = pltpu.async_copy(htab.at[idxs], rows2, sem_rows)
                dstloc = dstv - nbase
                dbase16 = dstloc * 16
                dbase = dstloc * HD
                cr.wait()
                for h in range(H):
                    el16v = plsc.load_gather(rows2, [iota, zero16 + (HD + h)])
                    er16v = plsc.load_gather(erloc, [dbase16 + h],
                                             mask=valid)
                    v = el16v + er16v
                    v = jnp.where(v >= 0.0, v, v * NEG)
                    w = jnp.exp(v)
                    plsc.addupdate_scatter(den1, [dbase16 + h], w, mask=valid)

                    @plsc.parallel_loop(0, D, 1, unroll=16)
                    def _dbody(dd):
                        col = h * D + dd
                        val = plsc.load_gather(rows2, [iota, zero16 + col])
                        plsc.addupdate_scatter(acc1, [dbase + col],
                                               val * w, mask=valid)

            g_lo = (s0 // 16) * 16
            seg0 = (g_lo // SEGE) * SEGE
            nsegs = jnp.maximum((s1 - seg0 + SEGE - 1) // SEGE, 0)

            def seg_body(si, _):
                segb = seg0 + si * SEGE
                segbc = jnp.minimum(segb, E - SEGE)
                pltpu.sync_copy(srcs.at[pl.ds(segbc, SEGE)], segs)
                pltpu.sync_copy(dsts.at[pl.ds(segbc, SEGE)], segd)
                glo = jnp.maximum(g_lo, segb)
                ghi = jnp.minimum(s1, segb + SEGE)
                ng = jnp.maximum((ghi - glo + 15) // 16, 0)

                def g_body(gi, _):
                    gb = glo + gi * 16
                    off = gb - segbc
                    srcv = segs[pl.ds(off, 16)]
                    dstv = segd[pl.ds(off, 16)]
                    eid = gb + iota
                    valid = (eid >= s0) & (eid < s1)
                    proc(srcv, dstv, valid)
                    return 0

                lax.fori_loop(0, ng, g_body, 0)
                return 0

            lax.fori_loop(0, nsegs, seg_body, 0)

            def out_body(j, _):
                invs = []
                for h in range(H):
                    dv = plsc.load_gather(den1, [zero16 + (j * 16 + h)])
                    invs.append(1.0 / jnp.maximum(dv, 1e-9))
                for k in range(D // 16):
                    s = jnp.zeros((16,), jnp.float32)
                    for h in range(H):
                        o = h * D + k * 16
                        a = acc1[pl.ds(j * HD + o, 16)]
                        v = a * invs[h] + biasv[pl.ds(o, 16)]
                        s = s + jnp.maximum(v, 0.0)
                    acc1[pl.ds(j * D + k * 16, 16)] = s
                return 0

            lax.fori_loop(0, NPT, out_body, 0)
            pltpu.sync_copy(acc1.at[pl.ds(0, NPT * D)],
                            out.at[pl.ds(nbase * D, NPT * D)])
            return 0

        lax.fori_loop(0, 2, rr_body, 0)

    return pl.kernel(
        body,
        out_type=jax.ShapeDtypeStruct((NPAD * D,), jnp.float32),
        mesh=mesh,
        compiler_params=pltpu.CompilerParams(needs_layout_passes=False),
        scratch_types=[
            pltpu.VMEM((NPT * HD,), jnp.float32),
            pltpu.VMEM((16, HDP), jnp.float32),
            pltpu.VMEM((SEGE,), jnp.int32),
            pltpu.VMEM((SEGE,), jnp.int32),
            pltpu.VMEM((NPT * 16,), jnp.float32),
            pltpu.VMEM((NPT * 16,), jnp.float32),
            pltpu.VMEM((16,), jnp.int32),
            pltpu.VMEM((80,), jnp.int32),
            pltpu.VMEM((HD,), jnp.float32),
            pltpu.SemaphoreType.DMA,
        ],
    )


_sc_gat_l1 = _sc_gat_edge(10, 64, 768)
_sc_gat_l2 = _sc_gat_edge(1, 128, 256)


def _alpad(al, H, D):
    eye = jnp.eye(16, dtype=jnp.float32)[:H]
    return (al[:, :, None] * eye[:, None, :]).reshape(H * D, 16)


def _pack(h, el, HDP):
    pad = HDP - h.shape[1] - 16
    return jnp.concatenate(
        [h, el, jnp.zeros((N, pad), jnp.float32)], axis=1)


def _padr(er):
    return jnp.concatenate(
        [er, jnp.zeros((NPAD - N, 16), jnp.float32)], axis=0).reshape(-1)


def _branch(x, src, dst, W1, al1, ar1, b1, W2, al2, ar2, b2):
    perm = jnp.argsort(dst)
    src_s = src[perm]
    dst_s = dst[perm]
    bounds = jnp.arange(0, NR + 1, dtype=jnp.int32) * NPT
    starts = jnp.searchsorted(dst_s, bounds).astype(jnp.int32)
    starts80 = jnp.zeros((80,), jnp.int32).at[: NR + 1].set(starts)

    h1 = x @ W1
    el1 = h1 @ _alpad(al1, 10, 64)
    er1 = h1 @ _alpad(ar1, 10, 64)
    hs1 = _sc_gat_l1(_pack(h1, el1, 768), _padr(er1),
                     src_s, dst_s, starts80, b1)
    x2 = hs1.reshape(NPAD, 64)[:N]

    h2 = x2 @ W2
    el2 = h2 @ _alpad(al2, 1, 128)
    er2 = h2 @ _alpad(ar2, 1, 128)
    ho2 = _sc_gat_l2(_pack(h2, el2, 256), _padr(er2),
                     src_s, dst_s, starts80, b2)
    return ho2.reshape(NPAD, 128)[:N]


_RB = 1000
_NB = N // _RB


def _readout_mlp_body(hlig_ref, hrec_ref, ohl_ref, ohr_ref,
                      w1_ref, b1_ref, w2_ref, b2_ref, out_ref,
                      accl_ref, accr_ref):
    i = pl.program_id(0)

    @pl.when(i == 0)
    def _init():
        accl_ref[...] = jnp.full((B, 128), -jnp.inf, jnp.float32)
        accr_ref[...] = jnp.full((B, 128), -jnp.inf, jnp.float32)

    def upd(acc_ref, h, oh):
        for g in range(B):
            m = oh[:, g][:, None] > 0
            v = jnp.max(jnp.where(m, h, -jnp.inf), axis=0)
            acc_ref[g, :] = jnp.maximum(acc_ref[g, :], v)

    upd(accl_ref, hlig_ref[...], ohl_ref[...])
    upd(accr_ref, hrec_ref[...], ohr_ref[...])

    @pl.when(i == _NB - 1)
    def _fin():
        sl = accl_ref[...]
        sl = jnp.where(sl > -1e30, sl, 0.0)
        sr = accr_ref[...]
        sr = jnp.where(sr > -1e30, sr, 0.0)
        hcat = jnp.concatenate([sl, sr], axis=1)
        z = jnp.maximum(jnp.dot(hcat, w1_ref[...],
                                preferred_element_type=jnp.float32)
                        + b1_ref[...], 0.0)
        z = jnp.maximum(jnp.dot(z, w2_ref[...],
                                preferred_element_type=jnp.float32)
                        + b2_ref[...], 0.0)
        out_ref[...] = z


def _readout_mlp(hlig, hrec, ohl, ohr, lin1_w, lin1_b, lin2_w, lin2_b):
    return pl.pallas_call(
        _readout_mlp_body,
        grid=(_NB,),
        in_specs=[
            pl.BlockSpec((_RB, 128), lambda i: (i, 0)),
            pl.BlockSpec((_RB, 128), lambda i: (i, 0)),
            pl.BlockSpec((_RB, B), lambda i: (i, 0)),
            pl.BlockSpec((_RB, B), lambda i: (i, 0)),
            pl.BlockSpec((256, 128), lambda i: (0, 0)),
            pl.BlockSpec((1, 128), lambda i: (0, 0)),
            pl.BlockSpec((128, 1), lambda i: (0, 0)),
            pl.BlockSpec((1, 1), lambda i: (0, 0)),
        ],
        out_specs=pl.BlockSpec((B, 1), lambda i: (0, 0)),
        out_shape=jax.ShapeDtypeStruct((B, 1), jnp.float32),
        scratch_shapes=[pltpu.VMEM((B, 128), jnp.float32),
                        pltpu.VMEM((B, 128), jnp.float32)],
    )(hlig, hrec, ohl, ohr, lin1_w, lin1_b.reshape(1, -1), lin2_w,
      lin2_b.reshape(1, 1))


def kernel(lig_x, rec_x, lig_edge_index, rec_edge_index, lig_graph_ids,
           rec_graph_ids, W1l, al1l, ar1l, b1l, W2l, al2l, ar2l, b2l,
           W1r, al1r, ar1r, b1r, W2r, al2r, ar2r, b2r,
           lin1_w, lin1_b, lin2_w, lin2_b):
    ls = lig_edge_index[0].astype(jnp.int32)
    ld = lig_edge_index[1].astype(jnp.int32)
    rs = rec_edge_index[0].astype(jnp.int32)
    rd = rec_edge_index[1].astype(jnp.int32)

    hlig = _branch(lig_x, ls, ld, W1l, al1l, ar1l, b1l, W2l, al2l, ar2l, b2l)
    hrec = _branch(rec_x, rs, rd, W1r, al1r, ar1r, b1r, W2r, al2r, ar2r, b2r)

    ohl = jax.nn.one_hot(lig_graph_ids, B, dtype=jnp.float32)
    ohr = jax.nn.one_hot(rec_graph_ids, B, dtype=jnp.float32)
    out = _readout_mlp(hlig, hrec, ohl, ohr, lin1_w, lin1_b, lin2_w, lin2_b)
    return out.flatten()


# speedup vs baseline: 4.4037x; 1.0487x over previous
import functools

import jax
import jax.numpy as jnp
from jax import lax
from jax.experimental import pallas as pl
from jax.experimental.pallas import tpu as pltpu
from jax.experimental.pallas import tpu_sc as plsc

N = 10000
E = 160000
B = 8
NEG = 0.2

NPT = 40
NR = 256
NRPT = NR // 32
NPAD = NR * NPT
SEGE = 4096


def _sc_gat_edge(H, D, HDP, TR, S):
    HD = H * D
    mesh = plsc.VectorSubcoreMesh(core_axis_name="c", subcore_axis_name="s")

    def body(htab, er16p, srcs, dsts, starts, bias, out, *scr):
        acc1, rowsA, segs, segd, den1, erloc, idx1, startsv, biasv = scr[:9]
        sems = scr[9:]
        wid = lax.axis_index("s") * 2 + lax.axis_index("c")
        pltpu.sync_copy(starts, startsv)
        pltpu.sync_copy(bias, biasv)
        iota = lax.iota(jnp.int32, 16)
        zero16 = iota * 0

        def rr_body(rr, _):
            r = wid * NRPT + rr
            nbase = r * NPT

            s0 = jnp.max(plsc.load_gather(startsv, [zero16 + r]))
            s1 = jnp.max(plsc.load_gather(startsv, [zero16 + (r + 1)]))

            @plsc.parallel_loop(0, NPT * HD // 16, 1, unroll=8)
            def _zacc(i):
                acc1[pl.ds(i * 16, 16)] = jnp.zeros((16,), jnp.float32)

            @plsc.parallel_loop(0, NPT, 1, unroll=8)
            def _zden(i):
                den1[pl.ds(i * 16, 16)] = jnp.zeros((16,), jnp.float32)

            pltpu.sync_copy(er16p.at[pl.ds(nbase * 32, NPT * 32)], erloc)

            t_lo = (s0 // TR) * TR
            seg0 = (t_lo // SEGE) * SEGE
            nsegs = jnp.maximum((s1 - seg0 + SEGE - 1) // SEGE, 0)

            def seg_body(si, _):
                segb = seg0 + si * SEGE
                segbc = jnp.minimum(segb, E - SEGE)
                pltpu.sync_copy(srcs.at[pl.ds(segbc, SEGE)], segs)
                pltpu.sync_copy(dsts.at[pl.ds(segbc, SEGE)], segd)
                tglo = jnp.maximum(t_lo, segb)
                thi = jnp.minimum(s1, segb + SEGE)
                ntr = jnp.maximum((thi - tglo + TR - 1) // TR, 0)

                def issue(t):
                    tb = tglo + t * TR
                    off = tb - segbc
                    slot = t % S
                    sb = slot * TR

                    @plsc.parallel_loop(0, TR // 16, 1, unroll=1)
                    def _cpi(u):
                        idx1[pl.ds(sb + u * 16, 16)] = (
                            segs[pl.ds(off + u * 16, 16)])

                    for ss in range(S):
                        @pl.when(slot == ss)
                        def _():
                            pltpu.async_copy(
                                htab.at[idx1.at[pl.ds(ss * TR, TR)]],
                                rowsA.at[pl.ds(ss * TR, TR)], sems[ss])

                def wait(t):
                    slot = t % S
                    for ss in range(S):
                        @pl.when(slot == ss)
                        def _():
                            pltpu.make_async_copy(
                                htab.at[idx1.at[pl.ds(ss * TR, TR)]],
                                rowsA.at[pl.ds(ss * TR, TR)],
                                sems[ss]).wait()

                for ss in range(S - 1):
                    @pl.when(ss < ntr)
                    def _():
                        issue(ss)

                def t_body(t, _):
                    @pl.when(t + (S - 1) < ntr)
                    def _():
                        issue(t + (S - 1))
                    wait(t)
                    tb = tglo + t * TR
                    off = tb - segbc
                    sb = (t % S) * TR

                    def u_body(u, _):
                        dstv = segd[pl.ds(off + u * 16, 16)]
                        eid = tb + u * 16 + iota
                        valid = (eid >= s0) & (eid < s1)
                        dstloc = dstv - nbase
                        dbase16 = dstloc * 16
                        dbase32 = dstloc * 32
                        dbase = dstloc * HD
                        rowv = sb + u * 16 + iota
                        for h in range(H):
                            a1 = plsc.load_gather(
                                rowsA, [rowv, zero16 + (HD + h)])
                            a2 = plsc.load_gather(
                                rowsA, [rowv, zero16 + (HD + 16 + h)])
                            b1 = plsc.load_gather(erloc, [dbase32 + h],
                                                  mask=valid)
                            b2 = plsc.load_gather(erloc, [dbase32 + (16 + h)],
                                                  mask=valid)
                            t1 = a1 * b1
                            w = jnp.where(t1 >= 1.0, t1, a2 * b2)
                            plsc.addupdate_scatter(den1, [dbase16 + h], w,
                                                   mask=valid)

                            @plsc.parallel_loop(0, D, 1, unroll=16)
                            def _dbody(dd):
                                col = h * D + dd
                                val = plsc.load_gather(rowsA,
                                                       [rowv, zero16 + col])
                                plsc.addupdate_scatter(acc1, [dbase + col],
                                                       val * w, mask=valid)
                        return 0

                    lax.fori_loop(0, TR // 16, u_body, 0)
                    return 0

                lax.fori_loop(0, ntr, t_body, 0)
                return 0

            lax.fori_loop(0, nsegs, seg_body, 0)

            def out_body(j, _):
                invs = []
                for h in range(H):
                    dv = plsc.load_gather(den1, [zero16 + (j * 16 + h)])
                    dv = jnp.maximum(dv, 1e-9)
                    inv = 1.0 / dv
                    inv = inv * (2.0 - dv * inv)
                    inv = inv * (2.0 - dv * inv)
                    invs.append(inv)
                for k in range(D // 16):
                    s = jnp.zeros((16,), jnp.float32)
                    for h in range(H):
                        o = h * D + k * 16
                        a = acc1[pl.ds(j * HD + o, 16)]
                        v = a * invs[h] + biasv[pl.ds(o, 16)]
                        s = s + jnp.maximum(v, 0.0)
                    acc1[pl.ds(j * D + k * 16, 16)] = s
                return 0

            lax.fori_loop(0, NPT, out_body, 0)
            pltpu.sync_copy(acc1.at[pl.ds(0, NPT * D)],
                            out.at[pl.ds(nbase * D, NPT * D)])
            return 0

        lax.fori_loop(0, NRPT, rr_body, 0)

    return pl.kernel(
        body,
        out_type=jax.ShapeDtypeStruct((NPAD * D,), jnp.float32),
        mesh=mesh,
        compiler_params=pltpu.CompilerParams(needs_layout_passes=False),
        scratch_types=[
            pltpu.VMEM((NPT * HD,), jnp.float32),
            pltpu.VMEM((S * TR, HDP), jnp.float32),
            pltpu.VMEM((SEGE,), jnp.int32),
            pltpu.VMEM((SEGE,), jnp.int32),
            pltpu.VMEM((NPT * 16,), jnp.float32),
            pltpu.VMEM((NPT * 32,), jnp.float32),
            pltpu.VMEM((S * TR,), jnp.int32),
            pltpu.VMEM((264,), jnp.int32),
            pltpu.VMEM((HD,), jnp.float32),
        ] + [pltpu.SemaphoreType.DMA] * S,
    )


_sc_gat_l1 = _sc_gat_edge(10, 64, 768, 32, 3)
_sc_gat_l2 = _sc_gat_edge(1, 128, 256, 64, 3)


def _alpad(al, H, D):
    eye = jnp.eye(16, dtype=jnp.float32)[:H]
    return (al[:, :, None] * eye[:, None, :]).reshape(H * D, 16)


def _pack(h, el, HDP):
    pad = HDP - h.shape[1] - 32
    return jnp.concatenate(
        [h, jnp.exp(el), jnp.exp(NEG * el),
         jnp.zeros((N, pad), jnp.float32)], axis=1)


def _padr(er):
    er2 = jnp.concatenate([jnp.exp(er), jnp.exp(NEG * er)], axis=1)
    return jnp.concatenate(
        [er2, jnp.zeros((NPAD - N, 32), jnp.float32)], axis=0).reshape(-1)


def _branch(x, src, dst, W1, al1, ar1, b1, W2, al2, ar2, b2):
    perm = jnp.argsort(dst)
    src_s = src[perm]
    dst_s = dst[perm]
    bounds = jnp.arange(0, NR + 1, dtype=jnp.int32) * NPT
    starts = jnp.searchsorted(dst_s, bounds).astype(jnp.int32)
    starts80 = jnp.zeros((264,), jnp.int32).at[: NR + 1].set(starts)

    HI = jax.lax.Precision.HIGHEST
    h1 = x @ W1
    el1 = jnp.dot(h1, _alpad(al1, 10, 64), precision=HI)
    er1 = jnp.dot(h1, _alpad(ar1, 10, 64), precision=HI)
    hs1 = _sc_gat_l1(_pack(h1, el1, 768), _padr(er1),
                     src_s, dst_s, starts80, b1)
    x2 = hs1.reshape(NPAD, 64)[:N]

    h2 = x2 @ W2
    el2 = jnp.dot(h2, _alpad(al2, 1, 128), precision=HI)
    er2 = jnp.dot(h2, _alpad(ar2, 1, 128), precision=HI)
    ho2 = _sc_gat_l2(_pack(h2, el2, 256), _padr(er2),
                     src_s, dst_s, starts80, b2)
    return ho2.reshape(NPAD, 128)[:N]


_RB = 1000
_NB = N // _RB


def _readout_mlp_body(hlig_ref, hrec_ref, ohl_ref, ohr_ref,
                      w1_ref, b1_ref, w2_ref, b2_ref, out_ref,
                      accl_ref, accr_ref):
    i = pl.program_id(0)

    @pl.when(i == 0)
    def _init():
        accl_ref[...] = jnp.full((B, 128), -jnp.inf, jnp.float32)
        accr_ref[...] = jnp.full((B, 128), -jnp.inf, jnp.float32)

    def upd(acc_ref, h, oh):
        for g in range(B):
            m = oh[:, g][:, None] > 0
            v = jnp.max(jnp.where(m, h, -jnp.inf), axis=0)
            acc_ref[g, :] = jnp.maximum(acc_ref[g, :], v)

    upd(accl_ref, hlig_ref[...], ohl_ref[...])
    upd(accr_ref, hrec_ref[...], ohr_ref[...])

    @pl.when(i == _NB - 1)
    def _fin():
        sl = accl_ref[...]
        sl = jnp.where(sl > -1e30, sl, 0.0)
        sr = accr_ref[...]
        sr = jnp.where(sr > -1e30, sr, 0.0)
        hcat = jnp.concatenate([sl, sr], axis=1)
        z = jnp.maximum(jnp.dot(hcat, w1_ref[...],
                                preferred_element_type=jnp.float32)
                        + b1_ref[...], 0.0)
        z = jnp.maximum(jnp.dot(z, w2_ref[...],
                                preferred_element_type=jnp.float32)
                        + b2_ref[...], 0.0)
        out_ref[...] = z


def _readout_mlp(hlig, hrec, ohl, ohr, lin1_w, lin1_b, lin2_w, lin2_b):
    return pl.pallas_call(
        _readout_mlp_body,
        grid=(_NB,),
        in_specs=[
            pl.BlockSpec((_RB, 128), lambda i: (i, 0)),
            pl.BlockSpec((_RB, 128), lambda i: (i, 0)),
            pl.BlockSpec((_RB, B), lambda i: (i, 0)),
            pl.BlockSpec((_RB, B), lambda i: (i, 0)),
            pl.BlockSpec((256, 128), lambda i: (0, 0)),
            pl.BlockSpec((1, 128), lambda i: (0, 0)),
            pl.BlockSpec((128, 1), lambda i: (0, 0)),
            pl.BlockSpec((1, 1), lambda i: (0, 0)),
        ],
        out_specs=pl.BlockSpec((B, 1), lambda i: (0, 0)),
        out_shape=jax.ShapeDtypeStruct((B, 1), jnp.float32),
        scratch_shapes=[pltpu.VMEM((B, 128), jnp.float32),
                        pltpu.VMEM((B, 128), jnp.float32)],
    )(hlig, hrec, ohl, ohr, lin1_w, lin1_b.reshape(1, -1), lin2_w,
      lin2_b.reshape(1, 1))


def kernel(lig_x, rec_x, lig_edge_index, rec_edge_index, lig_graph_ids,
           rec_graph_ids, W1l, al1l, ar1l, b1l, W2l, al2l, ar2l, b2l,
           W1r, al1r, ar1r, b1r, W2r, al2r, ar2r, b2r,
           lin1_w, lin1_b, lin2_w, lin2_b):
    ls = lig_edge_index[0].astype(jnp.int32)
    ld = lig_edge_index[1].astype(jnp.int32)
    rs = rec_edge_index[0].astype(jnp.int32)
    rd = rec_edge_index[1].astype(jnp.int32)

    hlig = _branch(lig_x, ls, ld, W1l, al1l, ar1l, b1l, W2l, al2l, ar2l, b2l)
    hrec = _branch(rec_x, rs, rd, W1r, al1r, ar1r, b1r, W2r, al2r, ar2r, b2r)

    ohl = jax.nn.one_hot(lig_graph_ids, B, dtype=jnp.float32)
    ohr = jax.nn.one_hot(rec_graph_ids, B, dtype=jnp.float32)
    out = _readout_mlp(hlig, hrec, ohl, ohr, lin1_w, lin1_b, lin2_w, lin2_b)
    return out.flatten()
